# baseline (speedup 1.0000x reference)
import functools

import jax
import jax.numpy as jnp
from jax.experimental import pallas as pl
from jax.experimental.pallas import tpu as pltpu
from jax.experimental.pallas import tpu_sc as plsc

_N = 10000
_E = 320000
_F = 128

_K = 48
_WAVE = 6
_NWAVE = 72
_NCHUNK = _WAVE * _NWAVE
_EPER = _K * _NCHUNK
_EPAD = 16 * _EPER
_NPAD = 10240
_RPT = _NPAD // 16

_mesh = plsc.VectorSubcoreMesh(
    core_axis_name="c", subcore_axis_name="s", num_cores=2, num_subcores=16
)


def _spmm_body(fea_hbm, src_hbm, dst_hbm, val_hbm, zeros_hbm, out_hbm,
               src_sl, dst_sl, val_sl, rows, acc_sh,
               g0, g1, g2, g3, g4, g5, s0, s1, s2, s3, s4, s5,
               l0, l1, l2):
    cid = jax.lax.axis_index("c")
    sid = jax.lax.axis_index("s")
    gsem = (g0, g1, g2, g3, g4, g5)
    ssem = (s0, s1, s2, s3, s4, s5)
    lsem = (l0, l1, l2)
    rbase = sid * _RPT

    def zero_acc():
        pltpu.sync_copy(zeros_hbm.at[pl.ds(rbase, _RPT)],
                        acc_sh.at[pl.ds(rbase, _RPT)])

    def start_slab(j, p):
        sl = pl.ds(p * _WAVE, _WAVE)
        pltpu.async_copy(src_hbm.at[cid, sid, j], src_sl.at[sl], lsem[p])
        pltpu.async_copy(dst_hbm.at[cid, sid, j], dst_sl.at[sl], lsem[p])
        pltpu.async_copy(val_hbm.at[cid, sid, j], val_sl.at[sl], lsem[p])

    def wait_slab(j, p):
        sl = pl.ds(p * _WAVE, _WAVE)
        pltpu.make_async_copy(src_hbm.at[cid, sid, j], src_sl.at[sl],
                              lsem[p]).wait()
        pltpu.make_async_copy(dst_hbm.at[cid, sid, j], dst_sl.at[sl],
                              lsem[p]).wait()
        pltpu.make_async_copy(val_hbm.at[cid, sid, j], val_sl.at[sl],
                              lsem[p]).wait()

    def start_scatter(p, k, b):
        pltpu.async_copy(rows.at[b], acc_sh.at[dst_sl.at[p * _WAVE + k]],
                         ssem[b], add=True)

    def wait_scatter(b):
        pltpu.make_async_copy(rows.at[b], acc_sh.at[dst_sl.at[0]],
                              ssem[b]).wait()

    def run_hop(table, hop):
        def start_gather(p, k, b):
            pltpu.async_copy(table.at[src_sl.at[p * _WAVE + k]], rows.at[b],
                             gsem[b])

        def wait_gather(p, k, b):
            pltpu.make_async_copy(table.at[src_sl.at[p * _WAVE + k]],
                                  rows.at[b], gsem[b]).wait()

        def stage(j, w, k):
            i = j * _WAVE + k
            b = k
            wait_gather(w, k, b)

            kg = (k + 4) % _WAVE
            wg = (w + (k + 4) // _WAVE) % 3

            @pl.when(i + 4 < _NCHUNK)
            def _():
                start_gather(wg, kg, kg)

            @pl.loop(0, _K, unroll=2)
            def _edge(e):
                v = plsc.load_gather(
                    val_sl, [jnp.full((16,), w * _WAVE + k, jnp.int32),
                             jnp.full((16,), e, jnp.int32)])
                for c in range(8):
                    sl = pl.ds(c * 16, 16)
                    rows[b, e, sl] = rows[b, e, sl] * v

            start_scatter(w, k, b)

            @pl.when(i >= 1)
            def _():
                wait_scatter((k + _WAVE - 1) % _WAVE)

        def wave(j, w):
            stage(j, w, 0)
            stage(j, w, 1)

            @pl.when(j + 2 < _NWAVE)
            def _():
                start_slab(j + 2, (w + 2) % 3)

            @pl.when(j + 1 < _NWAVE)
            def _():
                wait_slab(j + 1, (w + 1) % 3)

            for kk in range(2, _WAVE):
                stage(j, w, kk)

        start_slab(0, 0)
        start_slab(1, 1)
        wait_slab(0, 0)
        for kk in range(4):
            start_gather(0, kk, kk)

        @pl.loop(0, _NWAVE // 3)
        def _super(jj):
            j = jj * 3
            wave(j, 0)
            wave(j + 1, 1)
            wave(j + 2, 2)

        wait_scatter((_NCHUNK - 1) % _WAVE)
        plsc.subcore_barrier()
        pltpu.sync_copy(acc_sh.at[pl.ds(rbase, _RPT)],
                        out_hbm.at[hop, cid, pl.ds(rbase, _RPT)])
        plsc.subcore_barrier()

    zero_acc()
    plsc.subcore_barrier()
    run_hop(fea_hbm, 0)
    zero_acc()
    plsc.subcore_barrier()
    run_hop(out_hbm.at[0, cid], 1)


_spmm_chain = functools.partial(
    pl.kernel,
    out_type=jax.ShapeDtypeStruct((2, 2, _NPAD, _F), jnp.float32),
    mesh=_mesh,
    compiler_params=pltpu.CompilerParams(needs_layout_passes=False),
    scratch_types=[
        pltpu.VMEM((3 * _WAVE, _K), jnp.int32),
        pltpu.VMEM((3 * _WAVE, _K), jnp.int32),
        pltpu.VMEM((3 * _WAVE, _K), jnp.float32),
        pltpu.VMEM((_WAVE, _K, _F), jnp.float32),
        pltpu.VMEM_SHARED((_NPAD, _F), jnp.float32),
    ] + [pltpu.SemaphoreType.DMA] * 15,
)(_spmm_body)


_BLK = 1000


def _dense_body(f_ref, x1_ref, x1t_ref, x2_ref, x2t_ref, w_ref, ae_ref, b_ref,
                o_ref):
    h = jnp.concatenate(
        [f_ref[...], x1_ref[...], x1t_ref[...], x2_ref[...], x2t_ref[...]],
        axis=1)
    w = w_ref[...] * ae_ref[...]
    z = jnp.dot(h.astype(jnp.bfloat16), w.astype(jnp.bfloat16),
                preferred_element_type=jnp.float32)
    o_ref[...] = jnp.maximum(z + b_ref[...], 0.0)


def _dense(fea, x1, x1t, x2, x2t, w, alpha_exp, b2):
    row_spec = pl.BlockSpec((_BLK, _F), lambda i: (i, 0))
    return pl.pallas_call(
        _dense_body,
        grid=(_N // _BLK,),
        in_specs=[
            row_spec, row_spec, row_spec, row_spec, row_spec,
            pl.BlockSpec((5 * _F, _F), lambda i: (0, 0)),
            pl.BlockSpec((5 * _F, 1), lambda i: (0, 0)),
            pl.BlockSpec((1, _F), lambda i: (0, 0)),
        ],
        out_specs=row_spec,
        out_shape=jax.ShapeDtypeStruct((_N, _F), jnp.float32),
    )(fea, x1, x1t, x2, x2t, w, alpha_exp, b2)


def kernel(fea, adj_tilde_indices, adj_tilde_values, adj_indices, adj_values,
           alpha, W, b):
    pad = _EPAD - _E
    i32 = jnp.int32
    shp = (2, 16, _NWAVE, _WAVE, _K)
    src_all = jnp.concatenate([
        adj_indices[1].astype(i32), jnp.zeros((pad,), i32),
        adj_tilde_indices[1].astype(i32), jnp.zeros((pad,), i32),
    ]).reshape(shp)
    dst_all = jnp.concatenate([
        adj_indices[0].astype(i32), jnp.zeros((pad,), i32),
        adj_tilde_indices[0].astype(i32), jnp.zeros((pad,), i32),
    ]).reshape(shp)
    zpad = jnp.zeros((pad,), jnp.float32)
    val_all = jnp.concatenate(
        [adj_values, zpad, adj_tilde_values, zpad]).reshape(shp)
    zeros = jnp.zeros((_NPAD, _F), jnp.float32)

    r = _spmm_chain(fea, src_all, dst_all, val_all, zeros)

    alpha_exp = jnp.repeat(alpha, _F)[:, None]
    return _dense(fea, r[0, 0, :_N], r[0, 1, :_N], r[1, 0, :_N], r[1, 1, :_N],
                  W, alpha_exp, b.reshape(1, _F))


# speedup vs baseline: 2.2967x; 2.2967x over previous
import functools

import jax
import jax.numpy as jnp
from jax.experimental import pallas as pl
from jax.experimental.pallas import tpu as pltpu
from jax.experimental.pallas import tpu_sc as plsc

_N = 10000
_E = 320000
_F = 128

_K = 64
_WAVE = 5
_NWAVE = 63
_NCHUNK = _WAVE * _NWAVE
_EPER = _K * _NCHUNK
_EPAD = 16 * _EPER
_NPAD = 10240
_RPT = _NPAD // 16

_mesh = plsc.VectorSubcoreMesh(
    core_axis_name="c", subcore_axis_name="s", num_cores=2, num_subcores=16
)


def _spmm_body(fea_hbm, src_hbm, dst_hbm, val_hbm, zeros_hbm, out_hbm,
               src_sl, dst_sl, val_sl, rows, acc_sh,
               g0, g1, g2, g3, g4, s0, s1, s2, s3, s4, l0, l1, l2):
    cid = jax.lax.axis_index("c")
    sid = jax.lax.axis_index("s")
    gsem = (g0, g1, g2, g3, g4)
    ssem = (s0, s1, s2, s3, s4)
    lsem = (l0, l1, l2)
    rbase = sid * _RPT

    def zero_acc():
        pltpu.sync_copy(zeros_hbm.at[pl.ds(rbase, _RPT)],
                        acc_sh.at[pl.ds(rbase, _RPT)])

    def start_slab(j, p):
        sl = pl.ds(p * _WAVE, _WAVE)
        pltpu.async_copy(src_hbm.at[cid, sid, j], src_sl.at[sl], lsem[p])
        pltpu.async_copy(dst_hbm.at[cid, sid, j], dst_sl.at[sl], lsem[p])
        pltpu.async_copy(val_hbm.at[cid, sid, j], val_sl.at[sl], lsem[p])

    def wait_slab(j, p):
        sl = pl.ds(p * _WAVE, _WAVE)
        pltpu.make_async_copy(src_hbm.at[cid, sid, j], src_sl.at[sl],
                              lsem[p]).wait()
        pltpu.make_async_copy(dst_hbm.at[cid, sid, j], dst_sl.at[sl],
                              lsem[p]).wait()
        pltpu.make_async_copy(val_hbm.at[cid, sid, j], val_sl.at[sl],
                              lsem[p]).wait()

    def start_scatter(p, k, b):
        pltpu.async_copy(rows.at[b], acc_sh.at[dst_sl.at[p * _WAVE + k]],
                         ssem[b], add=True)

    def wait_scatter(b):
        pltpu.make_async_copy(rows.at[b], acc_sh.at[dst_sl.at[0]],
                              ssem[b]).wait()

    def run_hop(table, hop):
        def start_gather(p, k, b):
            pltpu.async_copy(table.at[src_sl.at[p * _WAVE + k]], rows.at[b],
                             gsem[b])

        def wait_gather(p, k, b):
            pltpu.make_async_copy(table.at[src_sl.at[p * _WAVE + k]],
                                  rows.at[b], gsem[b]).wait()

        def stage(j, w, k):
            i = j * _WAVE + k
            b = k
            wait_gather(w, k, b)

            kg = (k + 3) % _WAVE
            wg = (w + (k + 3) // _WAVE) % 3

            @pl.when(i + 3 < _NCHUNK)
            def _():
                start_gather(wg, kg, kg)

            @pl.loop(0, _K, unroll=2)
            def _edge(e):
                v = plsc.load_gather(
                    val_sl, [jnp.full((16,), w * _WAVE + k, jnp.int32),
                             jnp.full((16,), e, jnp.int32)])
                for c in range(8):
                    sl = pl.ds(c * 16, 16)
                    rows[b, e, sl] = rows[b, e, sl] * v

            start_scatter(w, k, b)

            @pl.when(i >= 1)
            def _():
                wait_scatter((k + 4) % _WAVE)

        def wave(j, w):
            stage(j, w, 0)
            stage(j, w, 1)

            @pl.when(j + 2 < _NWAVE)
            def _():
                start_slab(j + 2, (w + 2) % 3)

            @pl.when(j + 1 < _NWAVE)
            def _():
                wait_slab(j + 1, (w + 1) % 3)

            stage(j, w, 2)
            stage(j, w, 3)
            stage(j, w, 4)

        start_slab(0, 0)
        start_slab(1, 1)
        wait_slab(0, 0)
        start_gather(0, 0, 0)
        start_gather(0, 1, 1)
        start_gather(0, 2, 2)

        @pl.loop(0, _NWAVE // 3)
        def _super(jj):
            j = jj * 3
            wave(j, 0)
            wave(j + 1, 1)
            wave(j + 2, 2)

        wait_scatter(4)
        plsc.subcore_barrier()
        pltpu.sync_copy(acc_sh.at[pl.ds(rbase, _RPT)],
                        out_hbm.at[hop, cid, pl.ds(rbase, _RPT)])
        plsc.subcore_barrier()

    zero_acc()
    plsc.subcore_barrier()
    run_hop(fea_hbm, 0)
    zero_acc()
    plsc.subcore_barrier()
    run_hop(out_hbm.at[0, cid], 1)


_spmm_chain = functools.partial(
    pl.kernel,
    out_type=jax.ShapeDtypeStruct((2, 2, _NPAD, _F), jnp.float32),
    mesh=_mesh,
    compiler_params=pltpu.CompilerParams(needs_layout_passes=False),
    scratch_types=[
        pltpu.VMEM((3 * _WAVE, _K), jnp.int32),
        pltpu.VMEM((3 * _WAVE, _K), jnp.int32),
        pltpu.VMEM((3 * _WAVE, _K), jnp.float32),
        pltpu.VMEM((_WAVE, _K, _F), jnp.float32),
        pltpu.VMEM_SHARED((_NPAD, _F), jnp.float32),
    ] + [pltpu.SemaphoreType.DMA] * 13,
)(_spmm_body)


_BLK = 1000


def _dense_body(f_ref, x1_ref, x1t_ref, x2_ref, x2t_ref, w_ref, ae_ref, b_ref,
                o_ref):
    h = jnp.concatenate(
        [f_ref[...], x1_ref[...], x1t_ref[...], x2_ref[...], x2t_ref[...]],
        axis=1)
    w = w_ref[...] * ae_ref[...]
    z = jnp.dot(h.astype(jnp.bfloat16), w.astype(jnp.bfloat16),
                preferred_element_type=jnp.float32)
    o_ref[...] = jnp.maximum(z + b_ref[...], 0.0)


def _dense(fea, x1, x1t, x2, x2t, w, alpha_exp, b2):
    row_spec = pl.BlockSpec((_BLK, _F), lambda i: (i, 0))
    return pl.pallas_call(
        _dense_body,
        grid=(_N // _BLK,),
        in_specs=[
            row_spec, row_spec, row_spec, row_spec, row_spec,
            pl.BlockSpec((5 * _F, _F), lambda i: (0, 0)),
            pl.BlockSpec((5 * _F, 1), lambda i: (0, 0)),
            pl.BlockSpec((1, _F), lambda i: (0, 0)),
        ],
        out_specs=row_spec,
        out_shape=jax.ShapeDtypeStruct((_N, _F), jnp.float32),
    )(fea, x1, x1t, x2, x2t, w, alpha_exp, b2)


def kernel(fea, adj_tilde_indices, adj_tilde_values, adj_indices, adj_values,
           alpha, W, b):
    pad = _EPAD - _E
    i32 = jnp.int32
    shp = (2, 16, _NWAVE, _WAVE, _K)
    src_all = jnp.concatenate([
        adj_indices[1].astype(i32), jnp.zeros((pad,), i32),
        adj_tilde_indices[1].astype(i32), jnp.zeros((pad,), i32),
    ]).reshape(shp)
    dst_all = jnp.concatenate([
        adj_indices[0].astype(i32), jnp.zeros((pad,), i32),
        adj_tilde_indices[0].astype(i32), jnp.zeros((pad,), i32),
    ]).reshape(shp)
    zpad = jnp.zeros((pad,), jnp.float32)
    val_all = jnp.concatenate(
        [adj_values, zpad, adj_tilde_values, zpad]).reshape(shp)
    zeros = jnp.zeros((_NPAD, _F), jnp.float32)

    r = _spmm_chain(fea, src_all, dst_all, val_all, zeros)

    alpha_exp = jnp.repeat(alpha, _F)[:, None]
    return _dense(fea, r[0, 0, :_N], r[0, 1, :_N], r[1, 0, :_N], r[1, 1, :_N],
                  W, alpha_exp, b.reshape(1, _F))
